# baseline (device time: 39039 ns/iter reference)
import jax
import jax.numpy as jnp
from jax import lax
from jax.experimental import pallas as pl
from jax.experimental.pallas import tpu as pltpu

N_CHUNKS = 8


def kernel(A, B):
    m, k = A.shape
    k2, n = B.shape
    assert k == k2
    assert m % N_CHUNKS == 0
    cm = m // N_CHUNKS

    def body(
        a_ref, b_ref, out_ref,
        sq_ref, rq_ref, ss_ref, rs_ref,
        dsend_sems, drecv_sems, ssend_sems, srecv_sems,
    ):
        my_x = lax.axis_index("x")
        my_y = lax.axis_index("y")
        peer = (my_x, 1 - my_y)

        barrier_sem = pltpu.get_barrier_semaphore()
        pl.semaphore_signal(
            barrier_sem, inc=1, device_id=peer,
            device_id_type=pl.DeviceIdType.MESH,
        )

        def data_rdma(j):
            sl = pl.ds(j * cm, cm)
            return pltpu.make_async_remote_copy(
                src_ref=sq_ref.at[sl, :],
                dst_ref=rq_ref.at[sl, :],
                send_sem=dsend_sems.at[j],
                recv_sem=drecv_sems.at[j],
                device_id=peer,
                device_id_type=pl.DeviceIdType.MESH,
            )

        def scale_rdma(j):
            sl = pl.ds(j, 1)
            return pltpu.make_async_remote_copy(
                src_ref=ss_ref.at[sl, :],
                dst_ref=rs_ref.at[sl, :],
                send_sem=ssend_sems.at[j],
                recv_sem=srecv_sems.at[j],
                device_id=peer,
                device_id_type=pl.DeviceIdType.MESH,
            )

        def compute_quant(j):
            sl = pl.ds(j * cm, cm)
            partial = jnp.dot(
                a_ref[sl, :], b_ref[...], preferred_element_type=jnp.float32
            )
            out_ref[sl, :] = partial
            amax = jnp.max(jnp.abs(partial))
            ss_ref[pl.ds(j, 1), :] = jnp.full((1, 128), amax / 127.0, jnp.float32)
            q = jnp.clip(jnp.round(partial * (127.0 / amax)), -127.0, 127.0)
            sq_ref[sl, :] = q.astype(jnp.int8)

        compute_quant(0)
        pl.semaphore_wait(barrier_sem, 1)
        scale_rdma(0).start()
        data_rdma(0).start()
        for j in range(1, N_CHUNKS):
            compute_quant(j)
            scale_rdma(j).start()
            data_rdma(j).start()

        for j in range(N_CHUNKS):
            sl = pl.ds(j * cm, cm)
            scale_rdma(j).wait()
            data_rdma(j).wait()
            s = rs_ref[j, 0]
            out_ref[sl, :] = out_ref[sl, :] + rq_ref[sl, :].astype(jnp.float32) * s

    return pl.pallas_call(
        body,
        out_shape=jax.ShapeDtypeStruct((m, n), jnp.float32),
        in_specs=[
            pl.BlockSpec(memory_space=pltpu.VMEM),
            pl.BlockSpec(memory_space=pltpu.VMEM),
        ],
        out_specs=pl.BlockSpec(memory_space=pltpu.VMEM),
        scratch_shapes=[
            pltpu.VMEM((m, n), jnp.int8),
            pltpu.VMEM((m, n), jnp.int8),
            pltpu.VMEM((N_CHUNKS, 128), jnp.float32),
            pltpu.VMEM((N_CHUNKS, 128), jnp.float32),
            pltpu.SemaphoreType.DMA((N_CHUNKS,)),
            pltpu.SemaphoreType.DMA((N_CHUNKS,)),
            pltpu.SemaphoreType.DMA((N_CHUNKS,)),
            pltpu.SemaphoreType.DMA((N_CHUNKS,)),
        ],
        compiler_params=pltpu.CompilerParams(collective_id=0),
    )(A, B)
